# baseline (speedup 1.0000x reference)
import functools

import jax
import jax.numpy as jnp
from jax import lax
from jax.experimental import pallas as pl
from jax.experimental.pallas import tpu as pltpu
from jax.experimental.pallas import tpu_sc as plsc

_NUM_FIELDS = 26
_FIELD_DIM = 100000
_EMBED_D = 16
_BATCH = 16384
_NW = 32
_CHUNK = 128
_NCHUNK = _BATCH // _CHUNK
_CHUNKS_PER_W = _NCHUNK // _NW


def _sc_body(emb_hbm, lin_hbm, idx_hbm, cross_out, lin_out,
             idx_v, rows_v, lin_v, cross_v, linout_v, sem, seml):
    wid = lax.axis_index("s") * 2 + lax.axis_index("c")

    def run_chunk(g, carry):
        g = pl.multiple_of(g, 1)
        pltpu.sync_copy(idx_hbm.at[g], idx_v)
        cps = [pltpu.async_copy(emb_hbm.at[idx_v.at[f]], rows_v.at[f], sem)
               for f in range(_NUM_FIELDS)]
        lps = [pltpu.async_copy(lin_hbm.at[idx_v.at[f]], lin_v.at[f], seml)
               for f in range(_NUM_FIELDS)]
        for c in cps:
            c.wait()
        for c in lps:
            c.wait()

        def body(b, carry2):
            acc = rows_v[0, b, :]
            accsq = acc * acc
            for f in range(1, _NUM_FIELDS):
                v = rows_v[f, b, :]
                acc = acc + v
                accsq = accsq + v * v
            cross_v[b, :] = 0.5 * (acc * acc - accsq)
            return carry2

        lax.fori_loop(0, _CHUNK, body, 0, unroll=2)

        for j in range(_CHUNK // 16):
            lacc = lin_v[0, pl.ds(j * 16, 16)]
            for f in range(1, _NUM_FIELDS):
                lacc = lacc + lin_v[f, pl.ds(j * 16, 16)]
            linout_v[pl.ds(j * 16, 16)] = lacc

        base = pl.multiple_of(g * _CHUNK, _CHUNK)
        pltpu.sync_copy(cross_v, cross_out.at[pl.ds(base, _CHUNK)])
        pltpu.sync_copy(linout_v, lin_out.at[pl.ds(base, _CHUNK)])
        return carry

    lax.fori_loop(wid * _CHUNKS_PER_W, (wid + 1) * _CHUNKS_PER_W,
                  run_chunk, 0)


@jax.jit
def _sc_gather_fm(emb_table, lin_flat, idx_arr):
    mesh = plsc.VectorSubcoreMesh(core_axis_name="c", subcore_axis_name="s")
    f32 = jnp.float32
    return pl.kernel(
        _sc_body,
        out_type=(
            jax.ShapeDtypeStruct((_BATCH, _EMBED_D), f32),
            jax.ShapeDtypeStruct((_BATCH,), f32),
        ),
        mesh=mesh,
        compiler_params=pltpu.CompilerParams(use_tc_tiling_on_sc=False),
        scratch_types=[
            pltpu.VMEM((_NUM_FIELDS, _CHUNK), jnp.int32),
            pltpu.VMEM((_NUM_FIELDS, _CHUNK, _EMBED_D), f32),
            pltpu.VMEM((_NUM_FIELDS, _CHUNK), f32),
            pltpu.VMEM((_CHUNK, _EMBED_D), f32),
            pltpu.VMEM((_CHUNK,), f32),
            pltpu.SemaphoreType.DMA,
            pltpu.SemaphoreType.DMA,
        ],
    )(emb_table, lin_flat, idx_arr)


def _tc_body(cross_ref, lin_ref, bn0g, bn0b, W1, b1, bn1g, bn1b,
             W2, b2, bn2g, bn2b, W3, b3, linb, y_ref):
    eps = 1e-5
    c = cross_ref[:]
    m0 = jnp.mean(c, axis=0, keepdims=True)
    v0 = jnp.mean((c - m0) ** 2, axis=0, keepdims=True)
    cn = (c - m0) * lax.rsqrt(v0 + eps) * bn0g[:] + bn0b[:]
    h = jnp.dot(cn, W1[:], preferred_element_type=jnp.float32) + b1[:]
    m1 = jnp.mean(h, axis=0, keepdims=True)
    v1 = jnp.mean((h - m1) ** 2, axis=0, keepdims=True)
    h = jnp.maximum((h - m1) * lax.rsqrt(v1 + eps) * bn1g[:] + bn1b[:], 0.0)
    h = jnp.dot(h, W2[:], preferred_element_type=jnp.float32) + b2[:]
    m2 = jnp.mean(h, axis=0, keepdims=True)
    v2 = jnp.mean((h - m2) ** 2, axis=0, keepdims=True)
    h = jnp.maximum((h - m2) * lax.rsqrt(v2 + eps) * bn2g[:] + bn2b[:], 0.0)
    out = jnp.dot(h, W3[:], preferred_element_type=jnp.float32) + b3[0, 0]
    y_ref[:] = jax.nn.sigmoid(lin_ref[:] + linb[0, 0] + out)


@jax.jit
def _tc_head(cross, lin2d, bn0g, bn0b, W1, b1, bn1g, bn1b,
             W2, b2, bn2g, bn2b, W3, b3, linb):
    return pl.pallas_call(
        _tc_body,
        out_shape=jax.ShapeDtypeStruct((_BATCH, 1), jnp.float32),
    )(cross, lin2d, bn0g, bn0b, W1, b1, bn1g, bn1b,
      W2, b2, bn2g, bn2b, W3, b3, linb)


def kernel(x, emb_table, lin_table, lin_bias, bn0_g, bn0_b, W1, b1,
           bn1_g, bn1_b, W2, b2, bn2_g, bn2_b, W3, b3):
    offsets = (jnp.arange(_NUM_FIELDS, dtype=x.dtype) * _FIELD_DIM)[None, :]
    xi = x + offsets
    idx_arr = xi.reshape(_NCHUNK, _CHUNK, _NUM_FIELDS).transpose(0, 2, 1)
    lin_flat = lin_table.reshape(-1)
    cross, lin_sum = _sc_gather_fm(emb_table, lin_flat, idx_arr)
    y = _tc_head(
        cross, lin_sum.reshape(_BATCH, 1),
        bn0_g.reshape(1, -1), bn0_b.reshape(1, -1),
        W1, b1.reshape(1, -1), bn1_g.reshape(1, -1), bn1_b.reshape(1, -1),
        W2, b2.reshape(1, -1), bn2_g.reshape(1, -1), bn2_b.reshape(1, -1),
        W3, b3.reshape(1, 1), lin_bias.reshape(1, 1),
    )
    return y.reshape(_BATCH)
